# baseline (speedup 1.0000x reference)
import functools
import numpy as np
import jax
import jax.numpy as jnp
from jax import lax
from jax.experimental import pallas as pl
from jax.experimental.pallas import tpu as pltpu
from jax.experimental.pallas import tpu_sc as plsc

Z = 32
KC = 1024

_ROWS_PER_BLOCK = 8


def _argmax_body(xt_ref, locs_ref, logits_ref, idx_ref):
    locs = locs_ref[...]
    logits = logits_ref[...]
    m2 = jnp.sum(locs * locs, axis=1, keepdims=True)
    bias = (-0.5) * m2 + float(Z) * logits
    pieces = []
    for r in range(xt_ref.shape[0]):
        xt = xt_ref[r]
        s = lax.dot_general(
            locs, xt, (((1,), (0,)), ((), ())),
            preferred_element_type=jnp.float32)
        s = s + bias
        mx = jnp.max(s, axis=0)
        kio = lax.broadcasted_iota(jnp.int32, s.shape, 0).astype(jnp.float32)
        cand = jnp.where(s == mx[None, :], kio, float(KC))
        pieces.append(jnp.min(cand, axis=0).astype(jnp.int32))
    idx_ref[...] = jnp.concatenate(pieces, axis=0)


def _compute_idx(xt, locs, logits):
    b, zd, hw = xt.shape
    r = _ROWS_PER_BLOCK
    return pl.pallas_call(
        _argmax_body,
        grid=(b // r,),
        in_specs=[
            pl.BlockSpec((r, zd, hw), lambda i: (i, 0, 0)),
            pl.BlockSpec((KC, zd), lambda i: (0, 0)),
            pl.BlockSpec((KC, 1), lambda i: (0, 0)),
        ],
        out_specs=pl.BlockSpec((r * hw,), lambda i: (i,)),
        out_shape=jax.ShapeDtypeStruct((b * hw,), jnp.int32),
    )(xt, locs, logits[:, None])


def _make_sc_gather(b_total, d):
    info = plsc.get_sparse_core_info()
    nc, ns = info.num_cores, info.num_subcores
    nw = nc * ns
    assert b_total % (8 * nw) == 0
    b_per_w = b_total // nw
    mesh = plsc.VectorSubcoreMesh(core_axis_name="c", subcore_axis_name="s")

    @functools.partial(
        pl.kernel,
        mesh=mesh,
        out_type=jax.ShapeDtypeStruct((b_total, d), jnp.float32),
        scratch_types=[
            pltpu.VMEM((b_per_w,), jnp.int32),
            pltpu.VMEM((b_per_w,), jnp.int32),
            pltpu.VMEM((b_per_w, d), jnp.float32),
            pltpu.VMEM_SHARED((KC, d), jnp.float32),
            pltpu.VMEM_SHARED((b_total,), jnp.int32),
            pltpu.SemaphoreType.DMA,
        ],
        compiler_params=pltpu.CompilerParams(use_tc_tiling_on_sc=False),
    )
    def gather_kernel(table_hbm, idx_hbm, perm_hbm, out_hbm,
                      perm_v, idxp_v, rows_v, table_sh, idx_sh, sem):
        cid = lax.axis_index("c")
        sid = lax.axis_index("s")
        wid = sid * nc + cid
        base = wid * b_per_w

        @pl.when(sid == 0)
        def _():
            pltpu.sync_copy(table_hbm, table_sh)
            pltpu.sync_copy(idx_hbm, idx_sh)

        pltpu.sync_copy(perm_hbm.at[pl.ds(base, b_per_w)], perm_v)
        plsc.subcore_barrier()
        pltpu.async_copy(idx_sh.at[perm_v], idxp_v, sem).wait()
        pltpu.async_copy(table_sh.at[idxp_v], rows_v, sem).wait()
        pltpu.sync_copy(rows_v, out_hbm.at[pl.ds(base, b_per_w)])

    return gather_kernel


def _epilogue_body(in_ref, out_ref):
    x4 = in_ref[...]
    parts = [x4[:, 32 * m:32 * (m + 1)].T for m in range(4)]
    out_ref[0] = jnp.concatenate(parts, axis=1)


def _epilogue(out_v, b, hw, zd):
    rows = hw // 4
    return pl.pallas_call(
        _epilogue_body,
        grid=(b,),
        in_specs=[pl.BlockSpec((rows, 128), lambda i: (i, 0))],
        out_specs=pl.BlockSpec((1, zd, hw), lambda i: (i, 0, 0)),
        out_shape=jax.ShapeDtypeStruct((b, zd, hw), jnp.float32),
    )(out_v)


def kernel(x, locs, logits):
    b, hw, zd = x.shape
    xt = jnp.swapaxes(x, 1, 2)
    idx = _compute_idx(xt, locs, logits)
    pw = jnp.arange(hw, dtype=jnp.int32)
    perm = ((hw // 4) * (pw % 4) + pw // 4)[None, :] + (
        jnp.arange(b, dtype=jnp.int32) * hw)[:, None]
    out2d = _make_sc_gather(b * hw, zd)(locs, idx, perm.reshape(-1))
    out_t = _epilogue(out2d.reshape(b * hw * zd // 128, 128), b, hw, zd)
    return jnp.swapaxes(out_t, 1, 2)


# speedup vs baseline: 1.1052x; 1.1052x over previous
import functools
import numpy as np
import jax
import jax.numpy as jnp
from jax import lax
from jax.experimental import pallas as pl
from jax.experimental.pallas import tpu as pltpu
from jax.experimental.pallas import tpu_sc as plsc

Z = 32
KC = 1024

_ROWS_PER_BLOCK = 8


def _argmax_body(xt_ref, locs_ref, logits_ref, idx_ref):
    locs = locs_ref[...]
    logits = logits_ref[...]
    m2 = jnp.sum(locs * locs, axis=1, keepdims=True)
    bias = (-0.5) * m2 + float(Z) * logits
    pieces = []
    for r in range(xt_ref.shape[0]):
        xt = xt_ref[r]
        s = lax.dot_general(
            locs, xt, (((1,), (0,)), ((), ())),
            preferred_element_type=jnp.float32)
        s = s + bias
        mx = jnp.max(s, axis=0)
        kio = lax.broadcasted_iota(jnp.int32, s.shape, 0).astype(jnp.float32)
        cand = jnp.where(s == mx[None, :], kio, float(KC))
        pieces.append(jnp.min(cand, axis=0).astype(jnp.int32))
    idx_ref[...] = jnp.concatenate(pieces, axis=0)


def _compute_idx(xt, locs, logits):
    b, zd, hw = xt.shape
    r = _ROWS_PER_BLOCK
    return pl.pallas_call(
        _argmax_body,
        grid=(b // r,),
        in_specs=[
            pl.BlockSpec((r, zd, hw), lambda i: (i, 0, 0)),
            pl.BlockSpec((KC, zd), lambda i: (0, 0)),
            pl.BlockSpec((KC, 1), lambda i: (0, 0)),
        ],
        out_specs=pl.BlockSpec((r * hw,), lambda i: (i,)),
        out_shape=jax.ShapeDtypeStruct((b * hw,), jnp.int32),
    )(xt, locs, logits[:, None])


def _make_sc_gather(b_total, d):
    info = plsc.get_sparse_core_info()
    nc, ns = info.num_cores, info.num_subcores
    nw = nc * ns
    assert b_total % (8 * nw) == 0
    b_per_w = b_total // nw
    mesh = plsc.VectorSubcoreMesh(core_axis_name="c", subcore_axis_name="s")

    @functools.partial(
        pl.kernel,
        mesh=mesh,
        out_type=jax.ShapeDtypeStruct((b_total, d), jnp.float32),
        scratch_types=[
            pltpu.VMEM((b_per_w,), jnp.int32),
            pltpu.VMEM((b_per_w,), jnp.int32),
            pltpu.VMEM((b_per_w, d), jnp.float32),
            pltpu.VMEM_SHARED((KC, d), jnp.float32),
            pltpu.VMEM_SHARED((b_total,), jnp.int32),
            pltpu.SemaphoreType.DMA,
        ],
        compiler_params=pltpu.CompilerParams(use_tc_tiling_on_sc=False),
    )
    def gather_kernel(table_hbm, idx_hbm, perm_hbm, out_hbm,
                      perm_v, idxp_v, rows_v, table_sh, idx_sh, sem):
        cid = lax.axis_index("c")
        sid = lax.axis_index("s")
        wid = sid * nc + cid
        base = wid * b_per_w

        @pl.when(sid == 0)
        def _():
            pltpu.sync_copy(table_hbm, table_sh)
            pltpu.sync_copy(idx_hbm, idx_sh)

        pltpu.sync_copy(perm_hbm.at[pl.ds(base, b_per_w)], perm_v)
        plsc.subcore_barrier()
        pltpu.async_copy(idx_sh.at[perm_v], idxp_v, sem).wait()
        pltpu.async_copy(table_sh.at[idxp_v], rows_v, sem).wait()
        pltpu.sync_copy(rows_v, out_hbm.at[pl.ds(base, b_per_w)])

    return gather_kernel


_EPI_BATCH = 4


def _epilogue_body(in_ref, out_ref):
    rows = in_ref.shape[0] // _EPI_BATCH
    for g in range(_EPI_BATCH):
        x4 = in_ref[pl.ds(g * rows, rows), :]
        parts = [x4[:, 32 * m:32 * (m + 1)].T for m in range(4)]
        out_ref[g] = jnp.concatenate(parts, axis=1)


def _epilogue(out_v, b, hw, zd):
    rows = hw // 4
    g = _EPI_BATCH
    return pl.pallas_call(
        _epilogue_body,
        grid=(b // g,),
        in_specs=[pl.BlockSpec((rows * g, 128), lambda i: (i, 0))],
        out_specs=pl.BlockSpec((g, zd, hw), lambda i: (i, 0, 0)),
        out_shape=jax.ShapeDtypeStruct((b, zd, hw), jnp.float32),
    )(out_v)


def kernel(x, locs, logits):
    b, hw, zd = x.shape
    xt = jnp.swapaxes(x, 1, 2)
    idx = _compute_idx(xt, locs, logits)
    pw = jnp.arange(hw, dtype=jnp.int32)
    perm = ((hw // 4) * (pw % 4) + pw // 4)[None, :] + (
        jnp.arange(b, dtype=jnp.int32) * hw)[:, None]
    out2d = _make_sc_gather(b * hw, zd)(locs, idx, perm.reshape(-1))
    out_t = _epilogue(out2d.reshape(b * hw * zd // 128, 128), b, hw, zd)
    return jnp.swapaxes(out_t, 1, 2)


# speedup vs baseline: 1.1074x; 1.0019x over previous
import functools
import numpy as np
import jax
import jax.numpy as jnp
from jax import lax
from jax.experimental import pallas as pl
from jax.experimental.pallas import tpu as pltpu
from jax.experimental.pallas import tpu_sc as plsc

Z = 32
KC = 1024

_ROWS_PER_BLOCK = 8


def _argmax_body(xt_ref, locs_ref, logits_ref, idx_ref):
    locs = locs_ref[...]
    logits = logits_ref[...]
    m2 = jnp.sum(locs * locs, axis=1, keepdims=True)
    bias = (-0.5) * m2 + float(Z) * logits
    pieces = []
    for r in range(xt_ref.shape[0]):
        xt = xt_ref[r]
        s = lax.dot_general(
            locs, xt, (((1,), (0,)), ((), ())),
            preferred_element_type=jnp.float32)
        s = s + bias
        mx = jnp.max(s, axis=0)
        kio = lax.broadcasted_iota(jnp.int32, s.shape, 0).astype(jnp.float32)
        cand = jnp.where(s == mx[None, :], kio, float(KC))
        pieces.append(jnp.min(cand, axis=0).astype(jnp.int32))
    idx_ref[...] = jnp.concatenate(pieces, axis=0)


def _compute_idx(xt, locs, logits):
    b, zd, hw = xt.shape
    r = _ROWS_PER_BLOCK
    return pl.pallas_call(
        _argmax_body,
        grid=(b // r,),
        in_specs=[
            pl.BlockSpec((r, zd, hw), lambda i: (i, 0, 0)),
            pl.BlockSpec((KC, zd), lambda i: (0, 0)),
            pl.BlockSpec((KC, 1), lambda i: (0, 0)),
        ],
        out_specs=pl.BlockSpec((r * hw,), lambda i: (i,)),
        out_shape=jax.ShapeDtypeStruct((b * hw,), jnp.int32),
    )(xt, locs, logits[:, None])


def _make_sc_gather(b_total, d):
    info = plsc.get_sparse_core_info()
    nc, ns = info.num_cores, info.num_subcores
    nw = nc * ns
    assert b_total % (8 * nw) == 0
    b_per_w = b_total // nw
    mesh = plsc.VectorSubcoreMesh(core_axis_name="c", subcore_axis_name="s")

    @functools.partial(
        pl.kernel,
        mesh=mesh,
        out_type=jax.ShapeDtypeStruct((b_total, d), jnp.float32),
        scratch_types=[
            pltpu.VMEM((b_per_w,), jnp.int32),
            pltpu.VMEM((b_per_w,), jnp.int32),
            pltpu.VMEM((b_per_w, d), jnp.float32),
            pltpu.VMEM_SHARED((KC, d), jnp.float32),
            pltpu.VMEM_SHARED((b_total,), jnp.int32),
            pltpu.SemaphoreType.DMA,
        ],
        compiler_params=pltpu.CompilerParams(use_tc_tiling_on_sc=False),
    )
    def gather_kernel(table_hbm, idx_hbm, perm_hbm, out_hbm,
                      perm_v, idxp_v, rows_v, table_sh, idx_sh, sem):
        cid = lax.axis_index("c")
        sid = lax.axis_index("s")
        wid = sid * nc + cid
        base = wid * b_per_w

        @pl.when(sid == 0)
        def _():
            pltpu.sync_copy(table_hbm, table_sh)
            pltpu.sync_copy(idx_hbm, idx_sh)

        pltpu.sync_copy(perm_hbm.at[pl.ds(base, b_per_w)], perm_v)
        plsc.subcore_barrier()
        pltpu.async_copy(idx_sh.at[perm_v], idxp_v, sem).wait()
        pltpu.async_copy(table_sh.at[idxp_v], rows_v, sem).wait()
        pltpu.sync_copy(rows_v, out_hbm.at[pl.ds(base, b_per_w)])

    return gather_kernel


_EPI_BATCH = 8


def _epilogue_body(in_ref, out_ref):
    rows = in_ref.shape[0] // _EPI_BATCH
    for g in range(_EPI_BATCH):
        x4 = in_ref[pl.ds(g * rows, rows), :]
        parts = [x4[:, 32 * m:32 * (m + 1)].T for m in range(4)]
        out_ref[g] = jnp.concatenate(parts, axis=1)


def _epilogue(out_v, b, hw, zd):
    rows = hw // 4
    g = _EPI_BATCH
    return pl.pallas_call(
        _epilogue_body,
        grid=(b // g,),
        in_specs=[pl.BlockSpec((rows * g, 128), lambda i: (i, 0))],
        out_specs=pl.BlockSpec((g, zd, hw), lambda i: (i, 0, 0)),
        out_shape=jax.ShapeDtypeStruct((b, zd, hw), jnp.float32),
    )(out_v)


def kernel(x, locs, logits):
    b, hw, zd = x.shape
    xt = jnp.swapaxes(x, 1, 2)
    idx = _compute_idx(xt, locs, logits)
    pw = jnp.arange(hw, dtype=jnp.int32)
    perm = ((hw // 4) * (pw % 4) + pw // 4)[None, :] + (
        jnp.arange(b, dtype=jnp.int32) * hw)[:, None]
    out2d = _make_sc_gather(b * hw, zd)(locs, idx, perm.reshape(-1))
    out_t = _epilogue(out2d.reshape(b * hw * zd // 128, 128), b, hw, zd)
    return jnp.swapaxes(out_t, 1, 2)


# speedup vs baseline: 1.1095x; 1.0019x over previous
import functools
import numpy as np
import jax
import jax.numpy as jnp
from jax import lax
from jax.experimental import pallas as pl
from jax.experimental.pallas import tpu as pltpu
from jax.experimental.pallas import tpu_sc as plsc

Z = 32
KC = 1024

_ROWS_PER_BLOCK = 16


def _argmax_body(xt_ref, locs_ref, logits_ref, idx_ref):
    locs = locs_ref[...]
    logits = logits_ref[...]
    m2 = jnp.sum(locs * locs, axis=1, keepdims=True)
    bias = (-0.5) * m2 + float(Z) * logits
    pieces = []
    for r in range(xt_ref.shape[0]):
        xt = xt_ref[r]
        s = lax.dot_general(
            locs, xt, (((1,), (0,)), ((), ())),
            preferred_element_type=jnp.float32)
        s = s + bias
        mx = jnp.max(s, axis=0)
        kio = lax.broadcasted_iota(jnp.int32, s.shape, 0).astype(jnp.float32)
        cand = jnp.where(s == mx[None, :], kio, float(KC))
        pieces.append(jnp.min(cand, axis=0).astype(jnp.int32))
    idx_ref[...] = jnp.concatenate(pieces, axis=0)


def _compute_idx(xt, locs, logits):
    b, zd, hw = xt.shape
    r = _ROWS_PER_BLOCK
    return pl.pallas_call(
        _argmax_body,
        grid=(b // r,),
        in_specs=[
            pl.BlockSpec((r, zd, hw), lambda i: (i, 0, 0)),
            pl.BlockSpec((KC, zd), lambda i: (0, 0)),
            pl.BlockSpec((KC, 1), lambda i: (0, 0)),
        ],
        out_specs=pl.BlockSpec((r * hw,), lambda i: (i,)),
        out_shape=jax.ShapeDtypeStruct((b * hw,), jnp.int32),
    )(xt, locs, logits[:, None])


def _make_sc_gather(b_total, d):
    info = plsc.get_sparse_core_info()
    nc, ns = info.num_cores, info.num_subcores
    nw = nc * ns
    assert b_total % (8 * nw) == 0
    b_per_w = b_total // nw
    mesh = plsc.VectorSubcoreMesh(core_axis_name="c", subcore_axis_name="s")

    @functools.partial(
        pl.kernel,
        mesh=mesh,
        out_type=jax.ShapeDtypeStruct((b_total, d), jnp.float32),
        scratch_types=[
            pltpu.VMEM((b_per_w,), jnp.int32),
            pltpu.VMEM((b_per_w,), jnp.int32),
            pltpu.VMEM((b_per_w, d), jnp.float32),
            pltpu.VMEM_SHARED((KC, d), jnp.float32),
            pltpu.VMEM_SHARED((b_total,), jnp.int32),
            pltpu.SemaphoreType.DMA,
        ],
        compiler_params=pltpu.CompilerParams(use_tc_tiling_on_sc=False),
    )
    def gather_kernel(table_hbm, idx_hbm, perm_hbm, out_hbm,
                      perm_v, idxp_v, rows_v, table_sh, idx_sh, sem):
        cid = lax.axis_index("c")
        sid = lax.axis_index("s")
        wid = sid * nc + cid
        base = wid * b_per_w

        @pl.when(sid == 0)
        def _():
            pltpu.sync_copy(table_hbm, table_sh)
            pltpu.sync_copy(idx_hbm, idx_sh)

        pltpu.sync_copy(perm_hbm.at[pl.ds(base, b_per_w)], perm_v)
        plsc.subcore_barrier()
        pltpu.async_copy(idx_sh.at[perm_v], idxp_v, sem).wait()
        pltpu.async_copy(table_sh.at[idxp_v], rows_v, sem).wait()
        pltpu.sync_copy(rows_v, out_hbm.at[pl.ds(base, b_per_w)])

    return gather_kernel


_EPI_BATCH = 8


def _epilogue_body(in_ref, out_ref):
    rows = in_ref.shape[0] // _EPI_BATCH
    for g in range(_EPI_BATCH):
        x4 = in_ref[pl.ds(g * rows, rows), :]
        parts = [x4[:, 32 * m:32 * (m + 1)].T for m in range(4)]
        out_ref[g] = jnp.concatenate(parts, axis=1)


def _epilogue(out_v, b, hw, zd):
    rows = hw // 4
    g = _EPI_BATCH
    return pl.pallas_call(
        _epilogue_body,
        grid=(b // g,),
        in_specs=[pl.BlockSpec((rows * g, 128), lambda i: (i, 0))],
        out_specs=pl.BlockSpec((g, zd, hw), lambda i: (i, 0, 0)),
        out_shape=jax.ShapeDtypeStruct((b, zd, hw), jnp.float32),
    )(out_v)


def kernel(x, locs, logits):
    b, hw, zd = x.shape
    xt = jnp.swapaxes(x, 1, 2)
    idx = _compute_idx(xt, locs, logits)
    pw = jnp.arange(hw, dtype=jnp.int32)
    perm = ((hw // 4) * (pw % 4) + pw // 4)[None, :] + (
        jnp.arange(b, dtype=jnp.int32) * hw)[:, None]
    out2d = _make_sc_gather(b * hw, zd)(locs, idx, perm.reshape(-1))
    out_t = _epilogue(out2d.reshape(b * hw * zd // 128, 128), b, hw, zd)
    return jnp.swapaxes(out_t, 1, 2)
